# baseline (speedup 1.0000x reference)
import functools

import numpy as np
import jax
import jax.numpy as jnp
from jax import lax
from jax.experimental import pallas as pl
from jax.experimental.pallas import tpu as pltpu
from jax.experimental.pallas import tpu_sc as plsc

T = 4096
C = 1086
CP = 1152
OUT_T = 256
OUT_C = 454
BLK = 512
GRID = T // BLK
NW = 32
ROWS_PER = OUT_T // NW

_FACE = np.array([0, 6, 7, 11, 12, 13, 14, 15, 17, 22, 23, 24, 25, 26, 30, 31,
    33, 37, 38, 39, 40, 41, 42, 56, 61, 62, 72, 73, 74, 76, 77, 78, 80, 81, 82,
    84, 86, 87, 88, 89, 90, 91, 95, 96, 110, 112, 113, 122, 128, 130, 133, 144,
    145, 146, 153, 154, 155, 157, 158, 159, 160, 161, 163, 168, 173, 178, 179,
    180, 181, 183, 184, 185, 188, 189, 190, 191, 193, 196, 197, 232, 233, 243,
    244, 245, 246, 247, 249, 252, 253, 254, 255, 256, 259, 260, 263, 267, 268,
    269, 270, 271, 272, 286, 291, 292, 302, 303, 304, 306, 307, 308, 310, 311,
    312, 314, 316, 317, 318, 319, 320, 321, 324, 325, 339, 341, 351, 357, 359,
    362, 373, 374, 375, 380, 381, 382, 384, 385, 386, 387, 388, 390, 398, 402,
    403, 404, 405, 407, 408, 409, 412, 413, 414, 415, 417, 419, 453, 463, 464,
    465, 466, 467], dtype=np.int32)
_POSE = np.arange(489, 514, dtype=np.int32)
_LH = np.arange(468, 489, dtype=np.int32)
_RH = np.arange(522, 543, dtype=np.int32)
_HANDS = np.concatenate([_LH, _RH])
_LIDX = np.concatenate([_FACE, _POSE, _LH, _RH])

_PART_BOUNDS = [0, 160, 181, 206, 227]
_PART_LEN = [160, 21, 25, 21]


def _runs():
    runs = []
    start = prev = int(_LIDX[0])
    for li in _LIDX[1:]:
        li = int(li)
        if li == prev + 1:
            prev = li
        else:
            runs.append((2 * start, 2 * prev + 2))
            start = prev = li
    runs.append((2 * start, 2 * prev + 2))
    assert sum(b - a for a, b in runs) == OUT_C
    return runs


_RUNS = _runs()
SELW = 512


def _build_consts():
    S8 = np.zeros((8, OUT_C), np.float32)
    for u in range(227):
        p = next(k for k in range(4)
                 if _PART_BOUNDS[k] <= u < _PART_BOUNDS[k + 1])
        for ch in (0, 1):
            S8[2 * p + ch, 2 * u + ch] = 1.0
    NV = np.array([float(T) * _PART_LEN[p] for p in range(4) for _ in (0, 1)],
                  np.float32).reshape(8, 1)
    UT = np.triu(np.ones((BLK, BLK), np.float32))
    return S8, S8.T.copy(), NV, UT


_W8T, _S8T, _NV, _UT = _build_consts()
_HANDS_LO, _HANDS_HI = 370, 454


def _dot(a, b):
    return lax.dot_general(a, b, (((1,), (0,)), ((), ())),
                           precision=lax.Precision.HIGHEST,
                           preferred_element_type=jnp.float32)


def _dot_bf(a, b):
    return lax.dot_general(a, b, (((1,), (0,)), ((), ())),
                           precision=lax.Precision.DEFAULT,
                           preferred_element_type=jnp.float32)


def _reduce_body(xt_ref, ut_ref, colsum_ref, colsq_ref, cum_ref, xr_ref,
                 carry_ref):
    g = pl.program_id(0)
    xt = xt_ref[...]
    hs = jnp.sum(xt[_HANDS_LO:_HANDS_HI, :], axis=0, keepdims=True)
    m = (hs > 0.0).astype(jnp.float32)
    cumb = _dot_bf(m.astype(jnp.bfloat16), ut_ref[...])

    @pl.when(g == 0)
    def _():
        carry_ref[0] = 0.0

    carry = carry_ref[0]
    cum_ref[pl.ds(g, 1), :] = cumb + carry
    carry_ref[0] = carry + jnp.sum(m)

    xm = xt * m
    s = jnp.sum(xm, axis=1, keepdims=True)
    q = jnp.sum(xm * xt, axis=1, keepdims=True)

    @pl.when(g == 0)
    def _():
        colsum_ref[...] = s
        colsq_ref[...] = q

    @pl.when(g != 0)
    def _():
        colsum_ref[...] = colsum_ref[...] + s
        colsq_ref[...] = colsq_ref[...] + q

    xsel_t = jnp.concatenate(
        [xt, jnp.zeros((SELW - OUT_C, BLK), jnp.float32)], axis=0)
    xr_ref[...] = lax.transpose(xsel_t, (1, 0))


_reduce = pl.pallas_call(
    _reduce_body,
    grid=(GRID,),
    in_specs=[
        pl.BlockSpec((OUT_C, BLK), lambda g: (0, g)),
        pl.BlockSpec((BLK, BLK), lambda g: (0, 0)),
    ],
    out_specs=[
        pl.BlockSpec((OUT_C, 1), lambda g: (0, 0)),
        pl.BlockSpec((OUT_C, 1), lambda g: (0, 0)),
        pl.BlockSpec((GRID, BLK), lambda g: (0, 0)),
        pl.BlockSpec((BLK, SELW), lambda g: (g, 0)),
    ],
    out_shape=[
        jax.ShapeDtypeStruct((OUT_C, 1), jnp.float32),
        jax.ShapeDtypeStruct((OUT_C, 1), jnp.float32),
        jax.ShapeDtypeStruct((GRID, BLK), jnp.float32),
        jax.ShapeDtypeStruct((T, SELW), jnp.float32),
    ],
    scratch_shapes=[pltpu.SMEM((1,), jnp.float32)],
    compiler_params=pltpu.CompilerParams(
        dimension_semantics=("arbitrary",)),
)


@functools.cache
def _get_sc_route_gather():
    mesh = plsc.VectorSubcoreMesh(core_axis_name="c", subcore_axis_name="s")

    @functools.partial(
        pl.kernel,
        mesh=mesh,
        out_type=jax.ShapeDtypeStruct((OUT_T, SELW), jnp.float32),
        scratch_types=[
            pltpu.VMEM((T,), jnp.float32),
            pltpu.VMEM((16,), jnp.int32),
            pltpu.VMEM((ROWS_PER, SELW), jnp.float32),
            pltpu.SemaphoreType.DMA,
        ],
        compiler_params=pltpu.CompilerParams(needs_layout_passes=False),
    )
    def _sc_route_gather(cum_hbm, xrp_hbm, rows_out, cum_v, idx_v, rows_v,
                         sem):
        w = lax.axis_index("s") * 2 + lax.axis_index("c")
        pltpu.sync_copy(cum_hbm, cum_v)
        tail = cum_v[pl.ds(T - 16, 16)]
        total = tail[15]
        lane = lax.iota(jnp.int32, 16)
        j = lane & 7
        r1 = ((w * ROWS_PER + j) * 16 + 9).astype(jnp.float32)
        pos = jnp.zeros((16,), jnp.int32)
        for step in (2048, 1024, 512, 256, 128, 64, 32, 16, 8, 4, 2, 1):
            v = plsc.load_gather(cum_v, [pos + (step - 1)])
            pos = jnp.where(v < r1, pos + step, pos)
        posf = jnp.where(jnp.full((16,), total) >= r1, pos, 0)
        idx_v[...] = posf
        pltpu.async_copy(xrp_hbm.at[idx_v.at[pl.ds(0, ROWS_PER)]], rows_v,
                         sem).wait()
        pltpu.sync_copy(rows_v, rows_out.at[pl.ds(w * ROWS_PER, ROWS_PER)])

    return _sc_route_gather


def _final_body(rows_ref, colsum_ref, colsq_ref, col0_ref, cnt_ref,
                w8t_ref, s8t_ref, nv_ref, out_ref):
    fill = jnp.float32(T) - cnt_ref[0, 0]
    col0 = col0_ref[...]
    cs = colsum_ref[...] + fill * col0
    cq = colsq_ref[...] + fill * col0 * col0
    psum = _dot(w8t_ref[...], cs)
    psq = _dot(w8t_ref[...], cq)
    n = nv_ref[...]
    mean = psum / n
    var = jnp.maximum(psq / n - mean * mean, 0.0)
    std = jnp.sqrt(var)
    meanv = _dot(s8t_ref[...], mean)
    stdv = _dot(s8t_ref[...], std)
    selt = lax.transpose(rows_ref[...], (1, 0))[0:OUT_C, :]
    o = jnp.where(selt == 0.0, 0.0, (selt - meanv) / stdv)
    o = jnp.where(jnp.isnan(o), 0.0, o)
    out_ref[...] = o


_final = pl.pallas_call(
    _final_body,
    out_shape=jax.ShapeDtypeStruct((OUT_C, OUT_T), jnp.float32),
)


def kernel(data):
    xt3 = data.transpose(1, 2, 0)
    xsel = jnp.concatenate(
        [xt3[a // 2:b // 2, :, :] for a, b in _RUNS], axis=0
    ).reshape(OUT_C, T)
    colsum, colsq, cum2, xrp = _reduce(xsel, jnp.asarray(_UT, jnp.bfloat16))
    cumf = cum2.reshape(T)
    rows = _get_sc_route_gather()(cumf, xrp)
    outt = _final(rows, colsum, colsq, xsel[:, 0:1],
                  cumf[T - 1:].reshape(1, 1), jnp.asarray(_W8T),
                  jnp.asarray(_S8T), jnp.asarray(_NV))
    return outt.T, jnp.arange(OUT_T, dtype=jnp.float32)


# speedup vs baseline: 2.3795x; 2.3795x over previous
import functools

import numpy as np
import jax
import jax.numpy as jnp
from jax import lax
from jax.experimental import pallas as pl
from jax.experimental.pallas import tpu as pltpu
from jax.experimental.pallas import tpu_sc as plsc

T = 4096
C = 1086
CP = 1152
OUT_T = 256
OUT_C = 454
BLK = 512
GRID = T // BLK
NW = 32
ROWS_PER = OUT_T // NW

_FACE = np.array([0, 6, 7, 11, 12, 13, 14, 15, 17, 22, 23, 24, 25, 26, 30, 31,
    33, 37, 38, 39, 40, 41, 42, 56, 61, 62, 72, 73, 74, 76, 77, 78, 80, 81, 82,
    84, 86, 87, 88, 89, 90, 91, 95, 96, 110, 112, 113, 122, 128, 130, 133, 144,
    145, 146, 153, 154, 155, 157, 158, 159, 160, 161, 163, 168, 173, 178, 179,
    180, 181, 183, 184, 185, 188, 189, 190, 191, 193, 196, 197, 232, 233, 243,
    244, 245, 246, 247, 249, 252, 253, 254, 255, 256, 259, 260, 263, 267, 268,
    269, 270, 271, 272, 286, 291, 292, 302, 303, 304, 306, 307, 308, 310, 311,
    312, 314, 316, 317, 318, 319, 320, 321, 324, 325, 339, 341, 351, 357, 359,
    362, 373, 374, 375, 380, 381, 382, 384, 385, 386, 387, 388, 390, 398, 402,
    403, 404, 405, 407, 408, 409, 412, 413, 414, 415, 417, 419, 453, 463, 464,
    465, 466, 467], dtype=np.int32)
_POSE = np.arange(489, 514, dtype=np.int32)
_LH = np.arange(468, 489, dtype=np.int32)
_RH = np.arange(522, 543, dtype=np.int32)
_HANDS = np.concatenate([_LH, _RH])
_LIDX = np.concatenate([_FACE, _POSE, _LH, _RH])

_PART_BOUNDS = [0, 160, 181, 206, 227]
_PART_LEN = [160, 21, 25, 21]


def _runs():
    runs = []
    start = prev = int(_LIDX[0])
    for li in _LIDX[1:]:
        li = int(li)
        if li == prev + 1:
            prev = li
        else:
            runs.append((2 * start, 2 * prev + 2))
            start = prev = li
    runs.append((2 * start, 2 * prev + 2))
    assert sum(b - a for a, b in runs) == OUT_C
    return runs


_RUNS = _runs()
SELW = 512


def _build_consts():
    W8 = np.zeros((C, 8), np.float32)
    for p in range(4):
        for li in _LIDX[_PART_BOUNDS[p]:_PART_BOUNDS[p + 1]]:
            for ch in (0, 1):
                W8[2 * li + ch, 2 * p + ch] = 1.0
    S8 = np.zeros((8, OUT_C), np.float32)
    for u in range(227):
        p = next(k for k in range(4)
                 if _PART_BOUNDS[k] <= u < _PART_BOUNDS[k + 1])
        for ch in (0, 1):
            S8[2 * p + ch, 2 * u + ch] = 1.0
    NV = np.array([float(T) * _PART_LEN[p] for p in range(4) for _ in (0, 1)],
                  np.float32).reshape(8, 1)
    UT = np.triu(np.ones((BLK, BLK), np.float32))
    return W8.T.copy(), S8.T.copy(), NV, UT


_W8T, _S8T, _NV, _UT = _build_consts()


def _dot(a, b):
    return lax.dot_general(a, b, (((1,), (0,)), ((), ())),
                           precision=lax.Precision.HIGHEST,
                           preferred_element_type=jnp.float32)


def _dot_bf(a, b):
    return lax.dot_general(a, b, (((1,), (0,)), ((), ())),
                           precision=lax.Precision.DEFAULT,
                           preferred_element_type=jnp.float32)


def _reduce_body(xt_ref, ut_ref, colsum_ref, colsq_ref, cum_ref, xr_ref,
                 carry_ref):
    g = pl.program_id(0)
    xt = xt_ref[...]
    hs = (jnp.sum(xt[936:978, :], axis=0, keepdims=True)
          + jnp.sum(xt[1044:1086, :], axis=0, keepdims=True))
    m = (hs > 0.0).astype(jnp.float32)
    cumb = _dot_bf(m.astype(jnp.bfloat16), ut_ref[...])

    @pl.when(g == 0)
    def _():
        carry_ref[0] = 0.0

    carry = carry_ref[0]
    cum_ref[pl.ds(g, 1), :] = cumb + carry
    carry_ref[0] = carry + jnp.sum(m)

    xm = xt * m
    s = jnp.sum(xm, axis=1, keepdims=True)
    q = jnp.sum(xm * xt, axis=1, keepdims=True)

    @pl.when(g == 0)
    def _():
        colsum_ref[...] = s
        colsq_ref[...] = q

    @pl.when(g != 0)
    def _():
        colsum_ref[...] = colsum_ref[...] + s
        colsq_ref[...] = colsq_ref[...] + q

    parts = [xt[a:b, :] for a, b in _RUNS]
    parts.append(jnp.zeros((SELW - OUT_C, BLK), jnp.float32))
    xsel_t = jnp.concatenate(parts, axis=0)
    xr_ref[...] = lax.transpose(xsel_t, (1, 0))


_reduce = pl.pallas_call(
    _reduce_body,
    grid=(GRID,),
    in_specs=[
        pl.BlockSpec((C, BLK), lambda g: (0, g)),
        pl.BlockSpec((BLK, BLK), lambda g: (0, 0)),
    ],
    out_specs=[
        pl.BlockSpec((C, 1), lambda g: (0, 0)),
        pl.BlockSpec((C, 1), lambda g: (0, 0)),
        pl.BlockSpec((GRID, BLK), lambda g: (0, 0)),
        pl.BlockSpec((BLK, SELW), lambda g: (g, 0)),
    ],
    out_shape=[
        jax.ShapeDtypeStruct((C, 1), jnp.float32),
        jax.ShapeDtypeStruct((C, 1), jnp.float32),
        jax.ShapeDtypeStruct((GRID, BLK), jnp.float32),
        jax.ShapeDtypeStruct((T, SELW), jnp.float32),
    ],
    scratch_shapes=[pltpu.SMEM((1,), jnp.float32)],
    compiler_params=pltpu.CompilerParams(
        dimension_semantics=("arbitrary",)),
)


@functools.cache
def _get_sc_route_gather():
    mesh = plsc.VectorSubcoreMesh(core_axis_name="c", subcore_axis_name="s")

    @functools.partial(
        pl.kernel,
        mesh=mesh,
        out_type=jax.ShapeDtypeStruct((OUT_T, SELW), jnp.float32),
        scratch_types=[
            pltpu.VMEM((T,), jnp.float32),
            pltpu.VMEM((16,), jnp.int32),
            pltpu.VMEM((ROWS_PER, SELW), jnp.float32),
            pltpu.SemaphoreType.DMA,
        ],
        compiler_params=pltpu.CompilerParams(needs_layout_passes=False),
    )
    def _sc_route_gather(cum_hbm, xrp_hbm, rows_out, cum_v, idx_v, rows_v,
                         sem):
        w = lax.axis_index("s") * 2 + lax.axis_index("c")
        pltpu.sync_copy(cum_hbm, cum_v)
        tail = cum_v[pl.ds(T - 16, 16)]
        total = tail[15]
        lane = lax.iota(jnp.int32, 16)
        j = lane & 7
        r1 = ((w * ROWS_PER + j) * 16 + 9).astype(jnp.float32)
        pos = jnp.zeros((16,), jnp.int32)
        for step in (2048, 1024, 512, 256, 128, 64, 32, 16, 8, 4, 2, 1):
            v = plsc.load_gather(cum_v, [pos + (step - 1)])
            pos = jnp.where(v < r1, pos + step, pos)
        posf = jnp.where(jnp.full((16,), total) >= r1, pos, 0)
        idx_v[...] = posf
        pltpu.async_copy(xrp_hbm.at[idx_v.at[pl.ds(0, ROWS_PER)]], rows_v,
                         sem).wait()
        pltpu.sync_copy(rows_v, rows_out.at[pl.ds(w * ROWS_PER, ROWS_PER)])

    return _sc_route_gather


def _final_body(rows_ref, colsum_ref, colsq_ref, col0_ref, cnt_ref,
                w8t_ref, s8t_ref, nv_ref, out_ref):
    fill = jnp.float32(T) - cnt_ref[0, 0]
    col0 = col0_ref[...]
    cs = colsum_ref[...] + fill * col0
    cq = colsq_ref[...] + fill * col0 * col0
    psum = _dot(w8t_ref[...], cs)
    psq = _dot(w8t_ref[...], cq)
    n = nv_ref[...]
    mean = psum / n
    var = jnp.maximum(psq / n - mean * mean, 0.0)
    std = jnp.sqrt(var)
    meanv = _dot(s8t_ref[...], mean)
    stdv = _dot(s8t_ref[...], std)
    selt = lax.transpose(rows_ref[...], (1, 0))[0:OUT_C, :]
    o = jnp.where(selt == 0.0, 0.0, (selt - meanv) / stdv)
    o = jnp.where(jnp.isnan(o), 0.0, o)
    out_ref[...] = o


_final = pl.pallas_call(
    _final_body,
    out_shape=jax.ShapeDtypeStruct((OUT_C, OUT_T), jnp.float32),
)


def kernel(data):
    xt = data.transpose(1, 2, 0).reshape(C, T)
    colsum, colsq, cum2, xrp = _reduce(xt, jnp.asarray(_UT, jnp.bfloat16))
    cumf = cum2.reshape(T)
    rows = _get_sc_route_gather()(cumf, xrp)
    outt = _final(rows, colsum, colsq, xt[:, 0:1],
                  cumf[T - 1:].reshape(1, 1), jnp.asarray(_W8T),
                  jnp.asarray(_S8T), jnp.asarray(_NV))
    return outt.T, jnp.arange(OUT_T, dtype=jnp.float32)


# speedup vs baseline: 2.4478x; 1.0287x over previous
import functools

import numpy as np
import jax
import jax.numpy as jnp
from jax import lax
from jax.experimental import pallas as pl
from jax.experimental.pallas import tpu as pltpu
from jax.experimental.pallas import tpu_sc as plsc

T = 4096
C = 1086
CP = 1152
OUT_T = 256
OUT_C = 454
BLK = 1024
GRID = T // BLK
NW = 32
ROWS_PER = OUT_T // NW

_FACE = np.array([0, 6, 7, 11, 12, 13, 14, 15, 17, 22, 23, 24, 25, 26, 30, 31,
    33, 37, 38, 39, 40, 41, 42, 56, 61, 62, 72, 73, 74, 76, 77, 78, 80, 81, 82,
    84, 86, 87, 88, 89, 90, 91, 95, 96, 110, 112, 113, 122, 128, 130, 133, 144,
    145, 146, 153, 154, 155, 157, 158, 159, 160, 161, 163, 168, 173, 178, 179,
    180, 181, 183, 184, 185, 188, 189, 190, 191, 193, 196, 197, 232, 233, 243,
    244, 245, 246, 247, 249, 252, 253, 254, 255, 256, 259, 260, 263, 267, 268,
    269, 270, 271, 272, 286, 291, 292, 302, 303, 304, 306, 307, 308, 310, 311,
    312, 314, 316, 317, 318, 319, 320, 321, 324, 325, 339, 341, 351, 357, 359,
    362, 373, 374, 375, 380, 381, 382, 384, 385, 386, 387, 388, 390, 398, 402,
    403, 404, 405, 407, 408, 409, 412, 413, 414, 415, 417, 419, 453, 463, 464,
    465, 466, 467], dtype=np.int32)
_POSE = np.arange(489, 514, dtype=np.int32)
_LH = np.arange(468, 489, dtype=np.int32)
_RH = np.arange(522, 543, dtype=np.int32)
_HANDS = np.concatenate([_LH, _RH])
_LIDX = np.concatenate([_FACE, _POSE, _LH, _RH])

_PART_BOUNDS = [0, 160, 181, 206, 227]
_PART_LEN = [160, 21, 25, 21]


def _runs():
    runs = []
    start = prev = int(_LIDX[0])
    for li in _LIDX[1:]:
        li = int(li)
        if li == prev + 1:
            prev = li
        else:
            runs.append((2 * start, 2 * prev + 2))
            start = prev = li
    runs.append((2 * start, 2 * prev + 2))
    assert sum(b - a for a, b in runs) == OUT_C
    return runs


_RUNS = _runs()
SELW = 512


def _build_consts():
    W8 = np.zeros((C, 8), np.float32)
    for p in range(4):
        for li in _LIDX[_PART_BOUNDS[p]:_PART_BOUNDS[p + 1]]:
            for ch in (0, 1):
                W8[2 * li + ch, 2 * p + ch] = 1.0
    S8 = np.zeros((8, OUT_C), np.float32)
    for u in range(227):
        p = next(k for k in range(4)
                 if _PART_BOUNDS[k] <= u < _PART_BOUNDS[k + 1])
        for ch in (0, 1):
            S8[2 * p + ch, 2 * u + ch] = 1.0
    NV = np.array([float(T) * _PART_LEN[p] for p in range(4) for _ in (0, 1)],
                  np.float32).reshape(8, 1)
    UT = np.triu(np.ones((BLK, BLK), np.float32))
    return W8.T.copy(), S8.T.copy(), NV, UT


_W8T, _S8T, _NV, _UT = _build_consts()


def _dot(a, b):
    return lax.dot_general(a, b, (((1,), (0,)), ((), ())),
                           precision=lax.Precision.HIGHEST,
                           preferred_element_type=jnp.float32)


def _dot_bf(a, b):
    return lax.dot_general(a, b, (((1,), (0,)), ((), ())),
                           precision=lax.Precision.DEFAULT,
                           preferred_element_type=jnp.float32)


def _reduce_body(xt_ref, ut_ref, colsum_ref, colsq_ref, cum_ref, xr_ref,
                 carry_ref):
    g = pl.program_id(0)
    xt = xt_ref[...]
    hs = (jnp.sum(xt[936:978, :], axis=0, keepdims=True)
          + jnp.sum(xt[1044:1086, :], axis=0, keepdims=True))
    m = (hs > 0.0).astype(jnp.float32)
    cumb = _dot_bf(m.astype(jnp.bfloat16), ut_ref[...])

    @pl.when(g == 0)
    def _():
        carry_ref[0] = 0.0

    carry = carry_ref[0]
    cum_ref[pl.ds(g, 1), :] = cumb + carry
    carry_ref[0] = carry + jnp.sum(m)

    xm = xt * m
    s = jnp.sum(xm, axis=1, keepdims=True)
    q = jnp.sum(xm * xt, axis=1, keepdims=True)

    @pl.when(g == 0)
    def _():
        colsum_ref[...] = s
        colsq_ref[...] = q

    @pl.when(g != 0)
    def _():
        colsum_ref[...] = colsum_ref[...] + s
        colsq_ref[...] = colsq_ref[...] + q

    parts = [xt[a:b, :] for a, b in _RUNS]
    parts.append(jnp.zeros((SELW - OUT_C, BLK), jnp.float32))
    xsel_t = jnp.concatenate(parts, axis=0)
    xr_ref[...] = lax.transpose(xsel_t, (1, 0))


_reduce = pl.pallas_call(
    _reduce_body,
    grid=(GRID,),
    in_specs=[
        pl.BlockSpec((C, BLK), lambda g: (0, g)),
        pl.BlockSpec((BLK, BLK), lambda g: (0, 0)),
    ],
    out_specs=[
        pl.BlockSpec((C, 1), lambda g: (0, 0)),
        pl.BlockSpec((C, 1), lambda g: (0, 0)),
        pl.BlockSpec((GRID, BLK), lambda g: (0, 0)),
        pl.BlockSpec((BLK, SELW), lambda g: (g, 0)),
    ],
    out_shape=[
        jax.ShapeDtypeStruct((C, 1), jnp.float32),
        jax.ShapeDtypeStruct((C, 1), jnp.float32),
        jax.ShapeDtypeStruct((GRID, BLK), jnp.float32),
        jax.ShapeDtypeStruct((T, SELW), jnp.float32),
    ],
    scratch_shapes=[pltpu.SMEM((1,), jnp.float32)],
    compiler_params=pltpu.CompilerParams(
        dimension_semantics=("arbitrary",)),
)


@functools.cache
def _get_sc_route_gather():
    mesh = plsc.VectorSubcoreMesh(core_axis_name="c", subcore_axis_name="s")

    @functools.partial(
        pl.kernel,
        mesh=mesh,
        out_type=jax.ShapeDtypeStruct((OUT_T, SELW), jnp.float32),
        scratch_types=[
            pltpu.VMEM((T,), jnp.float32),
            pltpu.VMEM((16,), jnp.int32),
            pltpu.VMEM((ROWS_PER, SELW), jnp.float32),
            pltpu.SemaphoreType.DMA,
        ],
        compiler_params=pltpu.CompilerParams(needs_layout_passes=False),
    )
    def _sc_route_gather(cum_hbm, xrp_hbm, rows_out, cum_v, idx_v, rows_v,
                         sem):
        w = lax.axis_index("s") * 2 + lax.axis_index("c")
        pltpu.sync_copy(cum_hbm, cum_v)
        tail = cum_v[pl.ds(T - 16, 16)]
        total = tail[15]
        lane = lax.iota(jnp.int32, 16)
        j = lane & 7
        r1 = ((w * ROWS_PER + j) * 16 + 9).astype(jnp.float32)
        pos = jnp.zeros((16,), jnp.int32)
        for step in (2048, 1024, 512, 256, 128, 64, 32, 16, 8, 4, 2, 1):
            v = plsc.load_gather(cum_v, [pos + (step - 1)])
            pos = jnp.where(v < r1, pos + step, pos)
        posf = jnp.where(jnp.full((16,), total) >= r1, pos, 0)
        idx_v[...] = posf
        pltpu.async_copy(xrp_hbm.at[idx_v.at[pl.ds(0, ROWS_PER)]], rows_v,
                         sem).wait()
        pltpu.sync_copy(rows_v, rows_out.at[pl.ds(w * ROWS_PER, ROWS_PER)])

    return _sc_route_gather


def _final_body(rows_ref, colsum_ref, colsq_ref, col0_ref, cnt_ref,
                w8t_ref, s8t_ref, nv_ref, out_ref):
    fill = jnp.float32(T) - cnt_ref[0, 0]
    col0 = col0_ref[...]
    cs = colsum_ref[...] + fill * col0
    cq = colsq_ref[...] + fill * col0 * col0
    psum = _dot(w8t_ref[...], cs)
    psq = _dot(w8t_ref[...], cq)
    n = nv_ref[...]
    mean = psum / n
    var = jnp.maximum(psq / n - mean * mean, 0.0)
    std = jnp.sqrt(var)
    meanv = _dot(s8t_ref[...], mean)
    stdv = _dot(s8t_ref[...], std)
    selt = lax.transpose(rows_ref[...], (1, 0))[0:OUT_C, :]
    o = jnp.where(selt == 0.0, 0.0, (selt - meanv) / stdv)
    o = jnp.where(jnp.isnan(o), 0.0, o)
    out_ref[...] = o


_final = pl.pallas_call(
    _final_body,
    out_shape=jax.ShapeDtypeStruct((OUT_C, OUT_T), jnp.float32),
)


def kernel(data):
    xt = data.transpose(1, 2, 0).reshape(C, T)
    colsum, colsq, cum2, xrp = _reduce(xt, jnp.asarray(_UT, jnp.bfloat16))
    cumf = cum2.reshape(T)
    rows = _get_sc_route_gather()(cumf, xrp)
    outt = _final(rows, colsum, colsq, xt[:, 0:1],
                  cumf[T - 1:].reshape(1, 1), jnp.asarray(_W8T),
                  jnp.asarray(_S8T), jnp.asarray(_NV))
    return outt.T, jnp.arange(OUT_T, dtype=jnp.float32)
